# baseline (speedup 1.0000x reference)
import functools

import jax
import jax.numpy as jnp
from jax import lax
from jax.experimental import pallas as pl
from jax.experimental.pallas import tpu as pltpu
from jax.experimental.pallas import tpu_sc as plsc

N = 10000
E = 320000
D = 128
G = 64
NS = 16
NC = 2
W_WIN = 128
NWIN = 2560
EPAD = NWIN * W_WIN
WPT = NWIN // (NC * NS)
NBUF = 2
HWPT = WPT // 2
NP = 10240
TRASH = 10008
EPS = 1e-5

_mesh = plsc.VectorSubcoreMesh(core_axis_name="c", subcore_axis_name="s")


@functools.partial(
    pl.kernel,
    out_type=jax.ShapeDtypeStruct((NC, NP, D), jnp.float32),
    mesh=_mesh,
    scratch_types=[
        pltpu.VMEM_SHARED((NP, D), jnp.float32),
        pltpu.VMEM((W_WIN, D), jnp.float32),
        pltpu.VMEM((WPT, W_WIN), jnp.int32),
        pltpu.SemaphoreType.DMA,
    ],
)
def _deg_kernel(dst_hbm, zeros_hbm, out_hbm, acc, ones_v, di_all, sem):
    c = lax.axis_index("c")
    s = lax.axis_index("s")
    rpt = NP // NS
    pltpu.sync_copy(zeros_hbm.at[pl.ds(s * rpt, rpt)], acc.at[pl.ds(s * rpt, rpt)])
    pltpu.sync_copy(dst_hbm.at[pl.ds((c * NS + s) * WPT, WPT)], di_all)

    @pl.loop(0, W_WIN)
    def _(r):
        @pl.loop(0, D, step=16)
        def _(f):
            ones_v[r, pl.ds(f, 16)] = jnp.ones((16,), jnp.float32)

    plsc.subcore_barrier()

    @pl.loop(0, WPT)
    def _(w):
        pltpu.async_copy(ones_v, acc.at[di_all.at[w]], sem, add=True)

    @pl.loop(0, WPT)
    def _(w):
        pltpu.make_async_copy(ones_v, acc.at[di_all.at[w]], sem).wait()

    plsc.subcore_barrier()
    pltpu.sync_copy(acc.at[pl.ds(s * rpt, rpt)],
                    out_hbm.at[c].at[pl.ds(s * rpt, rpt)])


@functools.partial(
    pl.kernel,
    out_type=jax.ShapeDtypeStruct((NC, NP, D), jnp.float32),
    mesh=_mesh,
    scratch_types=[
        pltpu.VMEM_SHARED((NP, D), jnp.float32),
        pltpu.VMEM((NBUF, W_WIN, D), jnp.float32),
        pltpu.VMEM((HWPT, W_WIN), jnp.int32),
        pltpu.VMEM((HWPT, W_WIN), jnp.int32),
    ]
    + [pltpu.SemaphoreType.DMA] * (2 * NBUF),
)
def _agg_kernel(h2_hbm, src_hbm, dst_hbm, zeros_hbm, out_hbm,
                acc, rows_v, si_all, di_all, *sems):
    gsem = sems[:NBUF]
    ssem = sems[NBUF:]
    c = lax.axis_index("c")
    s = lax.axis_index("s")
    rpt = NP // NS
    pltpu.sync_copy(zeros_hbm.at[pl.ds(s * rpt, rpt)], acc.at[pl.ds(s * rpt, rpt)])
    plsc.subcore_barrier()

    def _gather(w, b):
        pltpu.async_copy(h2_hbm.at[si_all.at[w]], rows_v.at[b], gsem[b])

    def _scatter(w, b):
        pltpu.async_copy(rows_v.at[b], acc.at[di_all.at[w]], ssem[b], add=True)

    for half in range(WPT // HWPT):
        base = (c * NS + s) * WPT + half * HWPT
        pltpu.sync_copy(src_hbm.at[pl.ds(base, HWPT)], si_all)
        pltpu.sync_copy(dst_hbm.at[pl.ds(base, HWPT)], di_all)

        for b in range(NBUF):
            _gather(b, b)

        @pl.loop(0, HWPT // NBUF)
        def _(j):
            w0 = j * NBUF
            for b in range(NBUF):
                pltpu.make_async_copy(h2_hbm.at[si_all.at[w0 + b]],
                                      rows_v.at[b], gsem[b]).wait()
                _scatter(w0 + b, b)
            for b in range(NBUF):
                wn = w0 + NBUF + b

                @pl.when(wn < HWPT)
                def _():
                    pltpu.make_async_copy(rows_v.at[b], acc.at[di_all.at[w0 + b]],
                                          ssem[b]).wait()
                    _gather(wn, b)

        w_last = HWPT - NBUF
        for b in range(NBUF):
            pltpu.make_async_copy(rows_v.at[b], acc.at[di_all.at[w_last + b]],
                                  ssem[b]).wait()

    plsc.subcore_barrier()
    pltpu.sync_copy(acc.at[pl.ds(s * rpt, rpt)],
                    out_hbm.at[c].at[pl.ds(s * rpt, rpt)])


def _split_hi_lo(a):
    ai = lax.bitcast_convert_type(a, jnp.uint32)
    hi32 = lax.bitcast_convert_type(ai & jnp.uint32(0xFFFF0000), jnp.float32)
    lo = a - hi32
    return hi32.astype(jnp.bfloat16), lo.astype(jnp.bfloat16)


def _dot3(a, b):
    ah, al = _split_hi_lo(a)
    bh, bl = _split_hi_lo(b)
    d = lambda p, q: jnp.dot(p, q, preferred_element_type=jnp.float32)
    return d(ah, bh) + (d(ah, bl) + d(al, bh))


def _pre_body(x_ref, w_ref, dp_ref, h2_ref, dinv_ref):
    dp = dp_ref[...]
    deg = dp[0, :N, 0] + dp[1, :N, 0] + 1.0
    dinv = lax.rsqrt(deg)[:, None]
    h2 = _dot3(x_ref[...], w_ref[...])
    h2_ref[...] = h2 * dinv
    dinv_ref[...] = dinv


def _pre(x, w1, dp):
    return pl.pallas_call(
        _pre_body,
        out_shape=(jax.ShapeDtypeStruct((N, D), jnp.float32),
                   jax.ShapeDtypeStruct((N, 1), jnp.float32)),
    )(x, w1, dp)


def _norm_relu_bn(o_ref, h2_ref, dinv_ref, b_ref, g_ref, be_ref):
    o = o_ref[...]
    agg = o[0, :N] + o[1, :N]
    u = (agg + h2_ref[...]) * dinv_ref[...] + b_ref[...][None, :]
    u = jnp.maximum(u, 0.0)
    mu = jnp.mean(u, axis=0)
    du = u - mu
    var = jnp.mean(du * du, axis=0)
    return du * lax.rsqrt(var + EPS) * g_ref[...][None, :] + be_ref[...][None, :]


def _mid_body(o_ref, h2_ref, dinv_ref, b_ref, g_ref, be_ref, w_ref, out_ref):
    hn = _norm_relu_bn(o_ref, h2_ref, dinv_ref, b_ref, g_ref, be_ref)
    h2n = _dot3(hn, w_ref[...])
    out_ref[...] = h2n * dinv_ref[...]


def _mid(o, h2, dinv, b, g, be, wn):
    return pl.pallas_call(
        _mid_body,
        out_shape=jax.ShapeDtypeStruct((N, D), jnp.float32),
    )(o, h2, dinv, b, g, be, wn)


def _post_body(o_ref, h2_ref, dinv_ref, b_ref, g_ref, be_ref, batch_ref, out_ref):
    hn = _norm_relu_bn(o_ref, h2_ref, dinv_ref, b_ref, g_ref, be_ref)
    seg = lax.broadcasted_iota(jnp.int32, (G, N), 0)
    onehot_t = (seg == batch_ref[...][None, :]).astype(jnp.float32)
    out_ref[...] = _dot3(onehot_t, hn)


def _post(o, h2, dinv, b, g, be, batch):
    return pl.pallas_call(
        _post_body,
        out_shape=jax.ShapeDtypeStruct((G, D), jnp.float32),
    )(o, h2, dinv, b, g, be, batch)


def kernel(x, edge_index, batch, W1, b1, g1, be1, W2, b2, g2, be2,
           W3, b3, g3, be3, W4, b4, g4, be4):
    src = jnp.concatenate(
        [edge_index[0], jnp.zeros((EPAD - E,), jnp.int32)]).reshape(NWIN, W_WIN)
    dst = jnp.concatenate(
        [edge_index[1], jnp.full((EPAD - E,), TRASH, jnp.int32)]).reshape(NWIN, W_WIN)
    zeros_d = jnp.zeros((NP, D), jnp.float32)

    dp = _deg_kernel(dst, zeros_d)
    h2, dinv = _pre(x, W1, dp)
    for (b, g, be, wn) in ((b1, g1, be1, W2), (b2, g2, be2, W3),
                           (b3, g3, be3, W4)):
        o = _agg_kernel(h2, src, dst, zeros_d)
        h2 = _mid(o, h2, dinv, b, g, be, wn)
    o = _agg_kernel(h2, src, dst, zeros_d)
    return _post(o, h2, dinv, b4, g4, be4, batch)


# speedup vs baseline: 2.5981x; 2.5981x over previous
import functools

import jax
import jax.numpy as jnp
from jax import lax
from jax.experimental import pallas as pl
from jax.experimental.pallas import tpu as pltpu
from jax.experimental.pallas import tpu_sc as plsc

N = 10000
E = 320000
D = 128
G = 64
NS = 16
NC = 2
W_WIN = 128
NWIN = 2560
EPAD = NWIN * W_WIN
WPT = NWIN // (NC * NS)
NBUF = 2
HWPT = WPT // 2
NP = 10240
TRASH = 10008
EPS = 1e-5

_mesh = plsc.VectorSubcoreMesh(core_axis_name="c", subcore_axis_name="s")


@functools.partial(
    pl.kernel,
    out_type=jax.ShapeDtypeStruct((NC, NP, D), jnp.float32),
    mesh=_mesh,
    scratch_types=[
        pltpu.VMEM_SHARED((NP, D), jnp.float32),
        pltpu.VMEM((W_WIN, D), jnp.float32),
        pltpu.VMEM((WPT, W_WIN), jnp.int32),
        pltpu.SemaphoreType.DMA,
    ],
)
def _deg_kernel(dst_hbm, zeros_hbm, out_hbm, acc, ones_v, di_all, sem):
    c = lax.axis_index("c")
    s = lax.axis_index("s")
    rpt = NP // NS
    pltpu.sync_copy(zeros_hbm.at[pl.ds(s * rpt, rpt)], acc.at[pl.ds(s * rpt, rpt)])
    pltpu.sync_copy(dst_hbm.at[pl.ds((c * NS + s) * WPT, WPT)], di_all)

    @pl.loop(0, W_WIN)
    def _(r):
        @pl.loop(0, D, step=16)
        def _(f):
            ones_v[r, pl.ds(f, 16)] = jnp.ones((16,), jnp.float32)

    plsc.subcore_barrier()

    @pl.loop(0, WPT)
    def _(w):
        pltpu.async_copy(ones_v, acc.at[di_all.at[w]], sem, add=True)

    @pl.loop(0, WPT)
    def _(w):
        pltpu.make_async_copy(ones_v, acc.at[di_all.at[w]], sem).wait()

    plsc.subcore_barrier()
    pltpu.sync_copy(acc.at[pl.ds(s * rpt, rpt)],
                    out_hbm.at[c].at[pl.ds(s * rpt, rpt)])


@functools.partial(
    pl.kernel,
    out_type=jax.ShapeDtypeStruct((NC, NP, D), jnp.float32),
    mesh=_mesh,
    scratch_types=[
        pltpu.VMEM_SHARED((NP, D), jnp.float32),
        pltpu.VMEM((NBUF, W_WIN, D), jnp.float32),
        pltpu.VMEM((HWPT, W_WIN), jnp.int32),
        pltpu.VMEM((HWPT, W_WIN), jnp.int32),
    ]
    + [pltpu.SemaphoreType.DMA] * (2 * NBUF),
)
def _agg_kernel(h2_hbm, src_hbm, dst_hbm, zeros_hbm, out_hbm,
                acc, rows_v, si_all, di_all, *sems):
    gsem = sems[:NBUF]
    ssem = sems[NBUF:]
    c = lax.axis_index("c")
    s = lax.axis_index("s")
    rpt = NP // NS
    pltpu.sync_copy(zeros_hbm.at[pl.ds(s * rpt, rpt)], acc.at[pl.ds(s * rpt, rpt)])
    plsc.subcore_barrier()

    def _gather(w, b):
        pltpu.async_copy(h2_hbm.at[si_all.at[w]], rows_v.at[b], gsem[b])

    def _scatter(w, b):
        pltpu.async_copy(rows_v.at[b], acc.at[di_all.at[w]], ssem[b], add=True)

    for half in range(WPT // HWPT):
        base = (c * NS + s) * WPT + half * HWPT
        pltpu.sync_copy(src_hbm.at[pl.ds(base, HWPT)], si_all)
        pltpu.sync_copy(dst_hbm.at[pl.ds(base, HWPT)], di_all)

        for b in range(NBUF):
            _gather(b, b)

        @pl.loop(0, HWPT // NBUF)
        def _(j):
            w0 = j * NBUF
            for b in range(NBUF):
                pltpu.make_async_copy(h2_hbm.at[si_all.at[w0 + b]],
                                      rows_v.at[b], gsem[b]).wait()
                _scatter(w0 + b, b)
            for b in range(NBUF):
                wn = w0 + NBUF + b

                @pl.when(wn < HWPT)
                def _():
                    pltpu.make_async_copy(rows_v.at[b], acc.at[di_all.at[w0 + b]],
                                          ssem[b]).wait()
                    _gather(wn, b)

        w_last = HWPT - NBUF
        for b in range(NBUF):
            pltpu.make_async_copy(rows_v.at[b], acc.at[di_all.at[w_last + b]],
                                  ssem[b]).wait()

    plsc.subcore_barrier()
    pltpu.sync_copy(acc.at[pl.ds(s * rpt, rpt)],
                    out_hbm.at[c].at[pl.ds(s * rpt, rpt)])


def _split_hi_lo(a):
    ai = lax.bitcast_convert_type(a, jnp.uint32)
    hi32 = lax.bitcast_convert_type(ai & jnp.uint32(0xFFFF0000), jnp.float32)
    lo = a - hi32
    return hi32.astype(jnp.bfloat16), lo.astype(jnp.bfloat16)


def _dot3(a, b):
    ah, al = _split_hi_lo(a)
    bh, bl = _split_hi_lo(b)
    d = lambda p, q: jnp.dot(p, q, preferred_element_type=jnp.float32)
    return d(ah, bh) + (d(ah, bl) + d(al, bh))


def _pre_body(x_ref, w_ref, dp_ref, h2_ref, dinv_ref):
    dp = dp_ref[...]
    deg = dp[0, :N, 0] + dp[1, :N, 0] + 1.0
    dinv = lax.rsqrt(deg)[:, None]
    h2 = _dot3(x_ref[...], w_ref[...])
    h2_ref[...] = h2 * dinv
    dinv_ref[...] = dinv


def _pre(x, w1, dp):
    return pl.pallas_call(
        _pre_body,
        out_shape=(jax.ShapeDtypeStruct((N, D), jnp.float32),
                   jax.ShapeDtypeStruct((N, 1), jnp.float32)),
    )(x, w1, dp)


def _norm_relu_bn(o_ref, h2_ref, dinv_ref, b_ref, g_ref, be_ref):
    o = o_ref[...]
    agg = o[0, :N] + o[1, :N]
    u = (agg + h2_ref[...]) * dinv_ref[...] + b_ref[...][None, :]
    u = jnp.maximum(u, 0.0)
    mu = jnp.mean(u, axis=0)
    du = u - mu
    var = jnp.mean(du * du, axis=0)
    return du * lax.rsqrt(var + EPS) * g_ref[...][None, :] + be_ref[...][None, :]


def _mid_body(o_ref, h2_ref, dinv_ref, b_ref, g_ref, be_ref, w_ref, out_ref):
    hn = _norm_relu_bn(o_ref, h2_ref, dinv_ref, b_ref, g_ref, be_ref)
    h2n = _dot3(hn, w_ref[...])
    out_ref[...] = h2n * dinv_ref[...]


def _mid(o, h2, dinv, b, g, be, wn):
    return pl.pallas_call(
        _mid_body,
        out_shape=jax.ShapeDtypeStruct((N, D), jnp.float32),
    )(o, h2, dinv, b, g, be, wn)


def _post_body(o_ref, h2_ref, dinv_ref, b_ref, g_ref, be_ref, batch_ref, out_ref):
    hn = _norm_relu_bn(o_ref, h2_ref, dinv_ref, b_ref, g_ref, be_ref)
    seg = lax.broadcasted_iota(jnp.int32, (G, N), 0)
    onehot_t = (seg == batch_ref[...][None, :]).astype(jnp.float32)
    out_ref[...] = _dot3(onehot_t, hn)


def _post(o, h2, dinv, b, g, be, batch):
    return pl.pallas_call(
        _post_body,
        out_shape=jax.ShapeDtypeStruct((G, D), jnp.float32),
    )(o, h2, dinv, b, g, be, batch)


def kernel(x, edge_index, batch, W1, b1, g1, be1, W2, b2, g2, be2,
           W3, b3, g3, be3, W4, b4, g4, be4):
    pad_i = jnp.arange(EPAD - E, dtype=jnp.int32)
    src = jnp.concatenate(
        [edge_index[0], pad_i % N]).reshape(NWIN, W_WIN)
    dst = jnp.concatenate(
        [edge_index[1], N + (pad_i % (NP - N))]).reshape(NWIN, W_WIN)
    zeros_d = jnp.zeros((NP, D), jnp.float32)

    dp = _deg_kernel(dst, zeros_d)
    h2, dinv = _pre(x, W1, dp)
    for (b, g, be, wn) in ((b1, g1, be1, W2), (b2, g2, be2, W3),
                           (b3, g3, be3, W4)):
        o = _agg_kernel(h2, src, dst, zeros_d)
        h2 = _mid(o, h2, dinv, b, g, be, wn)
    o = _agg_kernel(h2, src, dst, zeros_d)
    return _post(o, h2, dinv, b4, g4, be4, batch)


# speedup vs baseline: 2.7853x; 1.0720x over previous
import dataclasses
import functools

import jax
import jax.numpy as jnp
from jax import lax
from jax.experimental import pallas as pl
from jax.experimental.pallas import tpu as pltpu
from jax.experimental.pallas import tpu_sc as plsc

N = 10000
E = 320000
D = 128
G = 64
NS = 16
NC = 2
W_WIN = 128
NWIN = 2560
EPAD = NWIN * W_WIN
WPT = NWIN // (NC * NS)
NBUF = 2
HWPT = WPT // 2
NP = 10240
TRASH = 10008
EPS = 1e-5

_mesh = plsc.VectorSubcoreMesh(core_axis_name="c", subcore_axis_name="s")

_cp = pltpu.CompilerParams()
if "needs_layout_passes" in pltpu.CompilerParams.__dataclass_fields__:
    _cp = dataclasses.replace(_cp, needs_layout_passes=False)


@functools.partial(
    pl.kernel,
    out_type=jax.ShapeDtypeStruct((NC * NS, 16, 1024), jnp.float32),
    mesh=_mesh,
    compiler_params=_cp,
    scratch_types=[
        pltpu.VMEM((16, 1024), jnp.float32),
        pltpu.VMEM((HWPT, W_WIN), jnp.int32),
    ],
)
def _deg_kernel(dst_hbm, out_hbm, dacc, di_all):
    c = lax.axis_index("c")
    s = lax.axis_index("s")
    wid = c * NS + s

    @pl.loop(0, 16)
    def _(r):
        @pl.loop(0, 1024, step=16)
        def _(q):
            dacc[r, pl.ds(q, 16)] = jnp.zeros((16,), jnp.float32)

    ones16 = jnp.ones((16,), jnp.float32)
    for half in range(WPT // HWPT):
        pltpu.sync_copy(
            dst_hbm.at[pl.ds(wid * WPT + half * HWPT, HWPT)], di_all)

        @pl.loop(0, HWPT)
        def _(w):
            @pl.loop(0, W_WIN, step=16)
            def _(k):
                idx = di_all[w, pl.ds(k, 16)]
                plsc.addupdate_scatter(
                    dacc, [idx >> 10, idx & 1023], ones16)

    pltpu.sync_copy(dacc, out_hbm.at[wid])


@functools.partial(
    pl.kernel,
    out_type=jax.ShapeDtypeStruct((NC, NP, D), jnp.float32),
    mesh=_mesh,
    scratch_types=[
        pltpu.VMEM_SHARED((NP, D), jnp.float32),
        pltpu.VMEM((NBUF, W_WIN, D), jnp.float32),
        pltpu.VMEM((HWPT, W_WIN), jnp.int32),
        pltpu.VMEM((HWPT, W_WIN), jnp.int32),
    ]
    + [pltpu.SemaphoreType.DMA] * (2 * NBUF),
)
def _agg_kernel(h2_hbm, src_hbm, dst_hbm, zeros_hbm, out_hbm,
                acc, rows_v, si_all, di_all, *sems):
    gsem = sems[:NBUF]
    ssem = sems[NBUF:]
    c = lax.axis_index("c")
    s = lax.axis_index("s")
    rpt = NP // NS
    pltpu.sync_copy(zeros_hbm.at[pl.ds(s * rpt, rpt)], acc.at[pl.ds(s * rpt, rpt)])
    plsc.subcore_barrier()

    def _gather(w, b):
        pltpu.async_copy(h2_hbm.at[si_all.at[w]], rows_v.at[b], gsem[b])

    def _scatter(w, b):
        pltpu.async_copy(rows_v.at[b], acc.at[di_all.at[w]], ssem[b], add=True)

    for half in range(WPT // HWPT):
        base = (c * NS + s) * WPT + half * HWPT
        pltpu.sync_copy(src_hbm.at[pl.ds(base, HWPT)], si_all)
        pltpu.sync_copy(dst_hbm.at[pl.ds(base, HWPT)], di_all)

        for b in range(NBUF):
            _gather(b, b)

        @pl.loop(0, HWPT // NBUF)
        def _(j):
            w0 = j * NBUF
            for b in range(NBUF):
                pltpu.make_async_copy(h2_hbm.at[si_all.at[w0 + b]],
                                      rows_v.at[b], gsem[b]).wait()
                _scatter(w0 + b, b)
            for b in range(NBUF):
                wn = w0 + NBUF + b

                @pl.when(wn < HWPT)
                def _():
                    pltpu.make_async_copy(rows_v.at[b], acc.at[di_all.at[w0 + b]],
                                          ssem[b]).wait()
                    _gather(wn, b)

        w_last = HWPT - NBUF
        for b in range(NBUF):
            pltpu.make_async_copy(rows_v.at[b], acc.at[di_all.at[w_last + b]],
                                  ssem[b]).wait()

    plsc.subcore_barrier()
    pltpu.sync_copy(acc.at[pl.ds(s * rpt, rpt)],
                    out_hbm.at[c].at[pl.ds(s * rpt, rpt)])


def _split_hi_lo(a):
    ai = lax.bitcast_convert_type(a, jnp.uint32)
    hi32 = lax.bitcast_convert_type(ai & jnp.uint32(0xFFFF0000), jnp.float32)
    lo = a - hi32
    return hi32.astype(jnp.bfloat16), lo.astype(jnp.bfloat16)


def _dot3(a, b):
    ah, al = _split_hi_lo(a)
    bh, bl = _split_hi_lo(b)
    d = lambda p, q: jnp.dot(p, q, preferred_element_type=jnp.float32)
    return d(ah, bh) + (d(ah, bl) + d(al, bh))


def _pre_body(x_ref, w_ref, dp_ref, h2_ref, dinv_ref):
    dp = dp_ref[...]
    deg = jnp.sum(dp[:, :N], axis=0) + 1.0
    dinv = lax.rsqrt(deg)[:, None]
    h2 = _dot3(x_ref[...], w_ref[...])
    h2_ref[...] = h2 * dinv
    dinv_ref[...] = dinv


def _pre(x, w1, dp):
    return pl.pallas_call(
        _pre_body,
        out_shape=(jax.ShapeDtypeStruct((N, D), jnp.float32),
                   jax.ShapeDtypeStruct((N, 1), jnp.float32)),
    )(x, w1, dp)


def _norm_relu_bn(o_ref, h2_ref, dinv_ref, b_ref, g_ref, be_ref):
    o = o_ref[...]
    agg = o[0, :N] + o[1, :N]
    u = (agg + h2_ref[...]) * dinv_ref[...] + b_ref[...][None, :]
    u = jnp.maximum(u, 0.0)
    mu = jnp.mean(u, axis=0)
    du = u - mu
    var = jnp.mean(du * du, axis=0)
    return du * lax.rsqrt(var + EPS) * g_ref[...][None, :] + be_ref[...][None, :]


def _mid_body(o_ref, h2_ref, dinv_ref, b_ref, g_ref, be_ref, w_ref, out_ref):
    hn = _norm_relu_bn(o_ref, h2_ref, dinv_ref, b_ref, g_ref, be_ref)
    h2n = _dot3(hn, w_ref[...])
    out_ref[...] = h2n * dinv_ref[...]


def _mid(o, h2, dinv, b, g, be, wn):
    return pl.pallas_call(
        _mid_body,
        out_shape=jax.ShapeDtypeStruct((N, D), jnp.float32),
    )(o, h2, dinv, b, g, be, wn)


def _post_body(o_ref, h2_ref, dinv_ref, b_ref, g_ref, be_ref, batch_ref, out_ref):
    hn = _norm_relu_bn(o_ref, h2_ref, dinv_ref, b_ref, g_ref, be_ref)
    seg = lax.broadcasted_iota(jnp.int32, (G, N), 0)
    onehot_t = (seg == batch_ref[...][None, :]).astype(jnp.float32)
    out_ref[...] = _dot3(onehot_t, hn)


def _post(o, h2, dinv, b, g, be, batch):
    return pl.pallas_call(
        _post_body,
        out_shape=jax.ShapeDtypeStruct((G, D), jnp.float32),
    )(o, h2, dinv, b, g, be, batch)


def kernel(x, edge_index, batch, W1, b1, g1, be1, W2, b2, g2, be2,
           W3, b3, g3, be3, W4, b4, g4, be4):
    pad_i = jnp.arange(EPAD - E, dtype=jnp.int32)
    src = jnp.concatenate(
        [edge_index[0], pad_i % N]).reshape(NWIN, W_WIN)
    dst = jnp.concatenate(
        [edge_index[1], N + (pad_i % (NP - N))]).reshape(NWIN, W_WIN)
    zeros_d = jnp.zeros((NP, D), jnp.float32)

    dp = _deg_kernel(dst).reshape(NC * NS, 16 * 1024)
    h2, dinv = _pre(x, W1, dp)
    for (b, g, be, wn) in ((b1, g1, be1, W2), (b2, g2, be2, W3),
                           (b3, g3, be3, W4)):
        o = _agg_kernel(h2, src, dst, zeros_d)
        h2 = _mid(o, h2, dinv, b, g, be, wn)
    o = _agg_kernel(h2, src, dst, zeros_d)
    return _post(o, h2, dinv, b4, g4, be4, batch)


# speedup vs baseline: 2.8332x; 1.0172x over previous
import dataclasses
import functools

import jax
import jax.numpy as jnp
from jax import lax
from jax.experimental import pallas as pl
from jax.experimental.pallas import tpu as pltpu
from jax.experimental.pallas import tpu_sc as plsc

N = 10000
E = 320000
D = 128
G = 64
NS = 16
NC = 2
W_WIN = 128
NWIN = 2560
EPAD = NWIN * W_WIN
WPT = NWIN // (NC * NS)
NBUF = 2
HWPT = WPT // 2
NP = 10240
TRASH = 10008
EPS = 1e-5

_mesh = plsc.VectorSubcoreMesh(core_axis_name="c", subcore_axis_name="s")

_cp = pltpu.CompilerParams()
if "needs_layout_passes" in pltpu.CompilerParams.__dataclass_fields__:
    _cp = dataclasses.replace(_cp, needs_layout_passes=False)


@functools.partial(
    pl.kernel,
    out_type=jax.ShapeDtypeStruct((NC * NS, 16, 1024), jnp.float32),
    mesh=_mesh,
    compiler_params=_cp,
    scratch_types=[
        pltpu.VMEM((16, 1024), jnp.float32),
        pltpu.VMEM((HWPT, W_WIN), jnp.int32),
    ],
)
def _deg_kernel(dst_hbm, out_hbm, dacc, di_all):
    c = lax.axis_index("c")
    s = lax.axis_index("s")
    wid = c * NS + s

    @pl.loop(0, 16)
    def _(r):
        @pl.loop(0, 1024, step=16)
        def _(q):
            dacc[r, pl.ds(q, 16)] = jnp.zeros((16,), jnp.float32)

    ones16 = jnp.ones((16,), jnp.float32)
    for half in range(WPT // HWPT):
        pltpu.sync_copy(
            dst_hbm.at[pl.ds(wid * WPT + half * HWPT, HWPT)], di_all)

        @pl.loop(0, HWPT)
        def _(w):
            @pl.loop(0, W_WIN, step=16)
            def _(k):
                idx = di_all[w, pl.ds(k, 16)]
                plsc.addupdate_scatter(
                    dacc, [idx >> 10, idx & 1023], ones16)

    pltpu.sync_copy(dacc, out_hbm.at[wid])


@functools.partial(
    pl.kernel,
    out_type=jax.ShapeDtypeStruct((NC, NP, D), jnp.float32),
    mesh=_mesh,
    scratch_types=[
        pltpu.VMEM_SHARED((NP, D), jnp.float32),
        pltpu.VMEM((NBUF, W_WIN, D), jnp.float32),
        pltpu.VMEM((HWPT, W_WIN), jnp.int32),
        pltpu.VMEM((HWPT, W_WIN), jnp.int32),
    ]
    + [pltpu.SemaphoreType.DMA] * (2 * NBUF),
)
def _agg_kernel(h2_hbm, src_hbm, dst_hbm, zeros_hbm, out_hbm,
                acc, rows_v, si_all, di_all, *sems):
    gsem = sems[:NBUF]
    ssem = sems[NBUF:]
    c = lax.axis_index("c")
    s = lax.axis_index("s")
    rpt = NP // NS

    def _gather(w, b):
        pltpu.async_copy(h2_hbm.at[si_all.at[w]], rows_v.at[b], gsem[b])

    def _scatter(w, b):
        pltpu.async_copy(rows_v.at[b], acc.at[di_all.at[w]], ssem[b], add=True)

    for half in range(WPT // HWPT):
        base = (c * NS + s) * WPT + half * HWPT
        pltpu.sync_copy(src_hbm.at[pl.ds(base, HWPT)], si_all)
        pltpu.sync_copy(dst_hbm.at[pl.ds(base, HWPT)], di_all)

        for b in range(NBUF):
            _gather(b, b)

        if half == 0:
            pltpu.sync_copy(zeros_hbm.at[pl.ds(s * rpt, rpt)],
                            acc.at[pl.ds(s * rpt, rpt)])
            plsc.subcore_barrier()

        @pl.loop(0, HWPT // NBUF)
        def _(j):
            w0 = j * NBUF
            for b in range(NBUF):
                pltpu.make_async_copy(h2_hbm.at[si_all.at[w0 + b]],
                                      rows_v.at[b], gsem[b]).wait()
                _scatter(w0 + b, b)
            for b in range(NBUF):
                wn = w0 + NBUF + b

                @pl.when(wn < HWPT)
                def _():
                    pltpu.make_async_copy(rows_v.at[b], acc.at[di_all.at[w0 + b]],
                                          ssem[b]).wait()
                    _gather(wn, b)

        w_last = HWPT - NBUF
        for b in range(NBUF):
            pltpu.make_async_copy(rows_v.at[b], acc.at[di_all.at[w_last + b]],
                                  ssem[b]).wait()

    plsc.subcore_barrier()
    pltpu.sync_copy(acc.at[pl.ds(s * rpt, rpt)],
                    out_hbm.at[c].at[pl.ds(s * rpt, rpt)])


def _split_hi_lo(a):
    ai = lax.bitcast_convert_type(a, jnp.uint32)
    hi32 = lax.bitcast_convert_type(ai & jnp.uint32(0xFFFF0000), jnp.float32)
    lo = a - hi32
    return hi32.astype(jnp.bfloat16), lo.astype(jnp.bfloat16)


def _dot3(a, b):
    ah, al = _split_hi_lo(a)
    bh, bl = _split_hi_lo(b)
    d = lambda p, q: jnp.dot(p, q, preferred_element_type=jnp.float32)
    return d(ah, bh) + (d(ah, bl) + d(al, bh))


def _pre_body(x_ref, w_ref, dp_ref, h2_ref, dinv_ref):
    dp = dp_ref[...]
    deg = jnp.sum(dp[:, :N], axis=0) + 1.0
    dinv = lax.rsqrt(deg)[:, None]
    h2 = _dot3(x_ref[...], w_ref[...])
    h2_ref[...] = h2 * dinv
    dinv_ref[...] = dinv


def _pre(x, w1, dp):
    return pl.pallas_call(
        _pre_body,
        out_shape=(jax.ShapeDtypeStruct((N, D), jnp.float32),
                   jax.ShapeDtypeStruct((N, 1), jnp.float32)),
    )(x, w1, dp)


def _norm_relu_bn(o_ref, h2_ref, dinv_ref, b_ref, g_ref, be_ref):
    o = o_ref[...]
    agg = o[0, :N] + o[1, :N]
    u = (agg + h2_ref[...]) * dinv_ref[...] + b_ref[...][None, :]
    u = jnp.maximum(u, 0.0)
    mu = jnp.mean(u, axis=0)
    du = u - mu
    var = jnp.mean(du * du, axis=0)
    return du * lax.rsqrt(var + EPS) * g_ref[...][None, :] + be_ref[...][None, :]


def _mid_body(o_ref, h2_ref, dinv_ref, b_ref, g_ref, be_ref, w_ref, out_ref):
    hn = _norm_relu_bn(o_ref, h2_ref, dinv_ref, b_ref, g_ref, be_ref)
    h2n = _dot3(hn, w_ref[...])
    out_ref[...] = h2n * dinv_ref[...]


def _mid(o, h2, dinv, b, g, be, wn):
    return pl.pallas_call(
        _mid_body,
        out_shape=jax.ShapeDtypeStruct((N, D), jnp.float32),
    )(o, h2, dinv, b, g, be, wn)


def _post_body(o_ref, h2_ref, dinv_ref, b_ref, g_ref, be_ref, batch_ref, out_ref):
    hn = _norm_relu_bn(o_ref, h2_ref, dinv_ref, b_ref, g_ref, be_ref)
    seg = lax.broadcasted_iota(jnp.int32, (G, N), 0)
    onehot_t = (seg == batch_ref[...][None, :]).astype(jnp.float32)
    out_ref[...] = _dot3(onehot_t, hn)


def _post(o, h2, dinv, b, g, be, batch):
    return pl.pallas_call(
        _post_body,
        out_shape=jax.ShapeDtypeStruct((G, D), jnp.float32),
    )(o, h2, dinv, b, g, be, batch)


def kernel(x, edge_index, batch, W1, b1, g1, be1, W2, b2, g2, be2,
           W3, b3, g3, be3, W4, b4, g4, be4):
    pad_i = jnp.arange(EPAD - E, dtype=jnp.int32)
    src = jnp.concatenate(
        [edge_index[0], pad_i % N]).reshape(NWIN, W_WIN)
    dst = jnp.concatenate(
        [edge_index[1], N + (pad_i % (NP - N))]).reshape(NWIN, W_WIN)
    zeros_d = jnp.zeros((NP, D), jnp.float32)

    dp = _deg_kernel(dst).reshape(NC * NS, 16 * 1024)
    h2, dinv = _pre(x, W1, dp)
    for (b, g, be, wn) in ((b1, g1, be1, W2), (b2, g2, be2, W3),
                           (b3, g3, be3, W4)):
        o = _agg_kernel(h2, src, dst, zeros_d)
        h2 = _mid(o, h2, dinv, b, g, be, wn)
    o = _agg_kernel(h2, src, dst, zeros_d)
    return _post(o, h2, dinv, b4, g4, be4, batch)
